# Initial kernel scaffold; baseline (speedup 1.0000x reference)
import jax
import jax.numpy as jnp

_B, _T, _C = 1, 2048, 1024
_H = 16
_D = _C // _H
_E = 8
_I = 2048

PREC = jax.lax.Precision.HIGHEST


def _ln(x, g, b, eps=1e-5):
    mu = jnp.mean(x, axis=-1, keepdims=True)
    var = jnp.mean((x - mu) ** 2, axis=-1, keepdims=True)
    return (x - mu) / jnp.sqrt(var + eps) * g + b


def _nrm(x, axis, eps=1e-12):
    n = jnp.sqrt(jnp.sum(x * x, axis=axis, keepdims=True))
    return x / jnp.maximum(n, eps)


def kernel(hidden_states, W_q, W_k, W_v, W_o, ln1_g, ln1_b, ln2_g, ln2_b,
           sim_matrix, threshold, w1, w2):
    B, T, C, H, D, E = _B, _T, _C, _H, _D, _E
    x = _ln(hidden_states, ln1_g, ln1_b)
    q = jnp.matmul(x, W_q.T, precision=PREC).reshape(B, T, H, D).transpose(0, 2, 1, 3)
    k = jnp.matmul(x, W_k.T, precision=PREC).reshape(B, T, H, D).transpose(0, 2, 1, 3)
    v = jnp.matmul(x, W_v.T, precision=PREC).reshape(B, T, H, D).transpose(0, 2, 1, 3)
    scores_att = jnp.einsum("bhtd,bhsd->bhts", q, k, precision=PREC) / jnp.sqrt(jnp.float32(D))
    causal = jnp.tril(jnp.ones((T, T), dtype=bool))
    scores_att = jnp.where(causal[None, None, :, :], scores_att, jnp.float32(-1e9))
    attn = jax.nn.softmax(scores_att, axis=-1)
    attn_out = jnp.einsum("bhts,bhsd->bhtd", attn, v, precision=PREC)
    attn_out = jnp.matmul(attn_out.transpose(0, 2, 1, 3).reshape(B, T, C), W_o.T, precision=PREC)
    h = hidden_states + attn_out

    x2 = _ln(h, ln2_g, ln2_b)
    flat = x2.reshape(-1, C)
    router_scores = jnp.matmul(_nrm(flat, -1), _nrm(sim_matrix, 0), precision=PREC)
    activated_mask = router_scores > threshold
    k_per_token = jnp.sum(activated_mask.astype(jnp.int32), axis=-1)

    a = jax.nn.gelu(jnp.einsum("nc,eic->nei", flat, w1, precision=PREC), approximate=False)
    dense_out = jnp.einsum("nei,eci->nec", a, w2, precision=PREC)
    expert_outputs_full = dense_out * activated_mask[:, :, None].astype(dense_out.dtype)

    ste_mask = activated_mask.astype(flat.dtype)
    weighted = expert_outputs_full * ste_mask[:, :, None]
    final = jnp.sum(weighted, axis=1)
    out_hidden = h + final.reshape(B, T, C)
    return (out_hidden, router_scores, expert_outputs_full, k_per_token)


# speedup vs baseline: 1.6201x; 1.6201x over previous
import functools

import jax
import jax.numpy as jnp
from jax.experimental import pallas as pl
from jax.experimental.pallas import tpu as pltpu

_B, _T, _C = 1, 2048, 1024
_H = 16
_D = _C // _H
_E = 8
_I = 2048

_TB1 = 512
_TB2 = 512
_TBM = 512


def _dot_t(a, b):
    return jax.lax.dot_general(a, b, (((1,), (1,)), ((), ())),
                               preferred_element_type=jnp.float32)


def _qkv_body(x_ref, wq_ref, wk_ref, wv_ref, q_ref, k_ref, v_ref):
    xb = x_ref[...].astype(jnp.bfloat16)
    q_ref[...] = _dot_t(xb, wq_ref[...])
    k_ref[...] = _dot_t(xb, wk_ref[...])
    v_ref[...] = _dot_t(xb, wv_ref[...])


def _router_body(attn_ref, hid_ref, wo_ref, g_ref, b_ref, sim_ref, thr_ref,
                 h_ref, x2_ref, sc_ref, mask_ref, kpt_ref):
    ao = attn_ref[...].astype(jnp.bfloat16)
    h = hid_ref[...] + _dot_t(ao, wo_ref[...])
    h_ref[...] = h
    mu = jnp.mean(h, axis=1, keepdims=True)
    var = jnp.mean((h - mu) ** 2, axis=1, keepdims=True)
    x2 = (h - mu) / jnp.sqrt(var + 1e-5) * g_ref[...] + b_ref[...]
    x2_ref[...] = x2.astype(jnp.bfloat16)
    n = jnp.sqrt(jnp.sum(x2 * x2, axis=1, keepdims=True))
    xn = x2 / jnp.maximum(n, 1e-12)
    sim = sim_ref[...]
    ns = jnp.sqrt(jnp.sum(sim * sim, axis=0, keepdims=True))
    sn = sim / jnp.maximum(ns, 1e-12)
    scores = jax.lax.dot_general(xn.astype(jnp.bfloat16), sn.astype(jnp.bfloat16),
                                 (((1,), (0,)), ((), ())),
                                 preferred_element_type=jnp.float32)
    sc_ref[...] = scores
    maskb = scores > thr_ref[0, 0]
    mask_ref[...] = maskb.astype(jnp.float32)
    kpt_ref[...] = jnp.sum(maskb.astype(jnp.int32), axis=1, keepdims=True)


def _moe_body(x2_ref, w1_ref, w2_ref, mask_ref, h_ref,
              eo_ref, oh_ref, acc_ref):
    e = pl.program_id(1)
    x = x2_ref[...]
    z = _dot_t(x, w1_ref[0])
    a = 0.5 * z * (1.0 + jax.lax.erf(z * 0.7071067811865476))
    o = _dot_t(a.astype(jnp.bfloat16), w2_ref[0])
    m = mask_ref[0, 0, :]
    om = o * m[:, None]
    eo_ref[...] = om

    @pl.when(e == 0)
    def _():
        acc_ref[...] = om

    @pl.when(e > 0)
    def _():
        acc_ref[...] += om

    @pl.when(e == _E - 1)
    def _():
        oh_ref[...] = h_ref[...] + acc_ref[...]


def kernel(hidden_states, W_q, W_k, W_v, W_o, ln1_g, ln1_b, ln2_g, ln2_b,
           sim_matrix, threshold, w1, w2):
    B, T, C, H, D, E, I = _B, _T, _C, _H, _D, _E, _I
    f32 = jnp.float32
    bf16 = jnp.bfloat16
    hid = hidden_states.reshape(T, C)

    mu1 = jnp.mean(hid, axis=-1, keepdims=True)
    var1 = jnp.mean((hid - mu1) ** 2, axis=-1, keepdims=True)
    x1 = (hid - mu1) / jnp.sqrt(var1 + 1e-5) * ln1_g + ln1_b
    q, k, v = pl.pallas_call(
        _qkv_body,
        grid=(T // _TB1,),
        in_specs=[
            pl.BlockSpec((_TB1, C), lambda i: (i, 0)),
            pl.BlockSpec((C, C), lambda i: (0, 0)),
            pl.BlockSpec((C, C), lambda i: (0, 0)),
            pl.BlockSpec((C, C), lambda i: (0, 0)),
        ],
        out_specs=[pl.BlockSpec((_TB1, C), lambda i: (i, 0))] * 3,
        out_shape=[jax.ShapeDtypeStruct((T, C), f32)] * 3,
    )(x1, W_q.astype(bf16), W_k.astype(bf16), W_v.astype(bf16))

    q4 = q.reshape(B, T, H, D).transpose(0, 2, 1, 3)
    k4 = k.reshape(B, T, H, D).transpose(0, 2, 1, 3)
    v4 = v.reshape(B, T, H, D).transpose(0, 2, 1, 3)
    scores_att = jnp.einsum("bhtd,bhsd->bhts", q4, k4) / jnp.sqrt(f32(D))
    causal = jnp.tril(jnp.ones((T, T), dtype=bool))
    scores_att = jnp.where(causal[None, None, :, :], scores_att, f32(-1e9))
    attn = jax.nn.softmax(scores_att, axis=-1)
    attn_out = jnp.einsum("bhts,bhsd->bhtd", attn, v4)
    attn_flat = attn_out.transpose(0, 2, 1, 3).reshape(T, C)

    h, x2bf, router_scores, maskf, kpt = pl.pallas_call(
        _router_body,
        grid=(T // _TB2,),
        in_specs=[
            pl.BlockSpec((_TB2, C), lambda i: (i, 0)),
            pl.BlockSpec((_TB2, C), lambda i: (i, 0)),
            pl.BlockSpec((C, C), lambda i: (0, 0)),
            pl.BlockSpec((1, C), lambda i: (0, 0)),
            pl.BlockSpec((1, C), lambda i: (0, 0)),
            pl.BlockSpec((C, E), lambda i: (0, 0)),
            pl.BlockSpec((1, 1), lambda i: (0, 0)),
        ],
        out_specs=[
            pl.BlockSpec((_TB2, C), lambda i: (i, 0)),
            pl.BlockSpec((_TB2, C), lambda i: (i, 0)),
            pl.BlockSpec((_TB2, E), lambda i: (i, 0)),
            pl.BlockSpec((_TB2, E), lambda i: (i, 0)),
            pl.BlockSpec((_TB2, 1), lambda i: (i, 0)),
        ],
        out_shape=[
            jax.ShapeDtypeStruct((T, C), f32),
            jax.ShapeDtypeStruct((T, C), bf16),
            jax.ShapeDtypeStruct((T, E), f32),
            jax.ShapeDtypeStruct((T, E), f32),
            jax.ShapeDtypeStruct((T, 1), jnp.int32),
        ],
    )(attn_flat, hid, W_o.astype(bf16), ln2_g.reshape(1, C),
      ln2_b.reshape(1, C), sim_matrix, threshold.reshape(1, 1))

    maskT3 = maskf.T.reshape(E, 1, T)

    eo, oh = pl.pallas_call(
        _moe_body,
        grid=(T // _TBM, E),
        in_specs=[
            pl.BlockSpec((_TBM, C), lambda i, e: (i, 0)),
            pl.BlockSpec((1, I, C), lambda i, e: (e, 0, 0)),
            pl.BlockSpec((1, C, I), lambda i, e: (e, 0, 0)),
            pl.BlockSpec((1, 1, _TBM), lambda i, e: (e, 0, i)),
            pl.BlockSpec((_TBM, C), lambda i, e: (i, 0)),
        ],
        out_specs=[
            pl.BlockSpec((_TBM, C), lambda i, e: (i, e)),
            pl.BlockSpec((_TBM, C), lambda i, e: (i, 0)),
        ],
        out_shape=[
            jax.ShapeDtypeStruct((T, E * C), f32),
            jax.ShapeDtypeStruct((T, C), f32),
        ],
        scratch_shapes=[pltpu.VMEM((_TBM, C), f32)],
    )(x2bf, w1.astype(bf16), w2.astype(bf16), maskT3, h)

    out_hidden = oh.reshape(B, T, C)
    expert_outputs_full = eo.reshape(T, E, C)
    k_per_token = kpt.reshape(T)
    return (out_hidden, router_scores, expert_outputs_full, k_per_token)
